# Initial kernel scaffold; baseline (speedup 1.0000x reference)
import jax
import jax.numpy as jnp
from jax.experimental import pallas as pl


def kernel(input, edge_index, W, b):
    raise NotImplementedError("write your pallas kernel here")


# speedup vs baseline: 5.3646x; 5.3646x over previous
import functools

import jax
import jax.numpy as jnp
from jax import lax
from jax.experimental import pallas as pl
from jax.experimental.pallas import tpu as pltpu
from jax.experimental.pallas import tpu_sc as plsc

N_NODES = 10000
N_EDGES = 320000
D = 128

NC = 2
NS = 16
NW = NC * NS
E_PER_TILE = N_EDGES // NW
CHUNK = 80
N_CHUNKS = E_PER_TILE // CHUNK
ACC_ROWS = 10240
ROWS_PER_TILE = ACC_ROWS // NS
ZROWS = 128


def _mm_body(x_ref, w_ref, o_ref):
    o_ref[...] = jnp.dot(x_ref[...], w_ref[...],
                         preferred_element_type=jnp.float32)


def _combine_body(p_ref, q_ref, b_ref, o_ref):
    o_ref[...] = p_ref[...] + q_ref[...] + b_ref[...]


def _sc_body(support_hbm, dst_hbm, src_hbm, out_hbm,
             src_v, dst_v, rows_v, zbuf_v, acc_sh, sem):
    c = lax.axis_index("c")
    s = lax.axis_index("s")
    wid = c * NS + s

    zero16 = jnp.zeros((16,), jnp.float32)

    def zstore(i, carry):
        zbuf_v[i // 8, pl.ds((i % 8) * 16, 16)] = zero16
        return carry

    lax.fori_loop(0, ZROWS * (D // 16), zstore, 0)
    for t in range(ROWS_PER_TILE // ZROWS):
        pltpu.sync_copy(zbuf_v,
                        acc_sh.at[pl.ds(s * ROWS_PER_TILE + t * ZROWS, ZROWS)])
    plsc.subcore_barrier()

    def chunk_body(j, carry):
        base = wid * E_PER_TILE + j * CHUNK
        pltpu.sync_copy(src_hbm.at[pl.ds(base, CHUNK)], src_v)
        cp = pltpu.async_copy(support_hbm.at[src_v], rows_v, sem)
        pltpu.sync_copy(dst_hbm.at[pl.ds(base, CHUNK)], dst_v)
        cp.wait()
        pltpu.sync_copy(rows_v, acc_sh.at[dst_v], add=True)
        return carry

    lax.fori_loop(0, N_CHUNKS, chunk_body, 0)
    plsc.subcore_barrier()

    for t in range(ROWS_PER_TILE // ZROWS):
        r = s * ROWS_PER_TILE + t * ZROWS
        pltpu.sync_copy(acc_sh.at[pl.ds(r, ZROWS)],
                        out_hbm.at[pl.ds(c * ACC_ROWS + r, ZROWS)])


_sc_aggregate = functools.partial(
    pl.kernel,
    out_type=jax.ShapeDtypeStruct((NC * ACC_ROWS, D), jnp.float32),
    mesh=plsc.VectorSubcoreMesh(core_axis_name="c", subcore_axis_name="s"),
    scratch_types=[
        pltpu.VMEM((CHUNK,), jnp.int32),
        pltpu.VMEM((CHUNK,), jnp.int32),
        pltpu.VMEM((CHUNK, D), jnp.float32),
        pltpu.VMEM((ZROWS, D), jnp.float32),
        pltpu.VMEM_SHARED((ACC_ROWS, D), jnp.float32),
        pltpu.SemaphoreType.DMA,
    ],
)(_sc_body)


@jax.jit
def kernel(input, edge_index, W, b):
    ei = edge_index.astype(jnp.int32)
    dst = ei[0]
    src = ei[1]

    support = pl.pallas_call(
        _mm_body,
        grid=(10,),
        in_specs=[
            pl.BlockSpec((1000, D), lambda i: (i, 0)),
            pl.BlockSpec((D, D), lambda i: (0, 0)),
        ],
        out_specs=pl.BlockSpec((1000, D), lambda i: (i, 0)),
        out_shape=jax.ShapeDtypeStruct((N_NODES, D), jnp.float32),
    )(input, W)

    partial = _sc_aggregate(support, dst, src)

    out = pl.pallas_call(
        _combine_body,
        grid=(125,),
        in_specs=[
            pl.BlockSpec((80, D), lambda i: (i, 0)),
            pl.BlockSpec((80, D), lambda i: (i + ACC_ROWS // 80, 0)),
            pl.BlockSpec((1, D), lambda i: (0, 0)),
        ],
        out_specs=pl.BlockSpec((80, D), lambda i: (i, 0)),
        out_shape=jax.ShapeDtypeStruct((N_NODES, D), jnp.float32),
    )(partial, partial, b.reshape(1, D))
    return out


# speedup vs baseline: 6.7921x; 1.2661x over previous
import functools

import jax
import jax.numpy as jnp
from jax import lax
from jax.experimental import pallas as pl
from jax.experimental.pallas import tpu as pltpu
from jax.experimental.pallas import tpu_sc as plsc

N_NODES = 10000
N_EDGES = 320000
D = 128

NC = 2
NS = 16
NW = NC * NS
E_PER_TILE = N_EDGES // NW
CHUNK = 80
N_CHUNKS = E_PER_TILE // CHUNK
ACC_ROWS = 10240
ROWS_PER_TILE = ACC_ROWS // NS
ZROWS = 128


def _mm_body(x_ref, w_ref, o_ref):
    o_ref[...] = jnp.dot(x_ref[...], w_ref[...],
                         preferred_element_type=jnp.float32)


def _combine_body(p_ref, q_ref, b_ref, o_ref):
    o_ref[...] = p_ref[...] + q_ref[...] + b_ref[...]


def _sc_body(support_hbm, dst_hbm, src_hbm, out_hbm,
             srcA, dstA, srcB, dstB, bufA, bufB, zbuf_v, acc_sh, semA, semB):
    c = lax.axis_index("c")
    s = lax.axis_index("s")
    wid = c * NS + s

    zero16 = jnp.zeros((16,), jnp.float32)

    def zstore(i, carry):
        zbuf_v[i // 8, pl.ds((i % 8) * 16, 16)] = zero16
        return carry

    lax.fori_loop(0, ZROWS * (D // 16), zstore, 0)
    for t in range(ROWS_PER_TILE // ZROWS):
        pltpu.sync_copy(zbuf_v,
                        acc_sh.at[pl.ds(s * ROWS_PER_TILE + t * ZROWS, ZROWS)])
    plsc.subcore_barrier()

    def gather(j, srcb, buf, sem):
        pltpu.sync_copy(src_hbm.at[pl.ds(wid * E_PER_TILE + j * CHUNK, CHUNK)],
                        srcb)
        pltpu.async_copy(support_hbm.at[srcb], buf, sem)

    def wait_scatter(j, srcb, dstb, buf, sem):
        pltpu.make_async_copy(support_hbm.at[srcb], buf, sem).wait()
        pltpu.sync_copy(dst_hbm.at[pl.ds(wid * E_PER_TILE + j * CHUNK, CHUNK)],
                        dstb)
        pltpu.sync_copy(buf, acc_sh.at[dstb], add=True)

    gather(0, srcA, bufA, semA)
    gather(1, srcB, bufB, semB)

    def pipe_body(i, carry):
        j = 2 * i
        wait_scatter(j, srcA, dstA, bufA, semA)
        gather(j + 2, srcA, bufA, semA)
        wait_scatter(j + 1, srcB, dstB, bufB, semB)
        gather(j + 3, srcB, bufB, semB)
        return carry

    lax.fori_loop(0, (N_CHUNKS - 3) // 2, pipe_body, 0)
    wait_scatter(N_CHUNKS - 3, srcA, dstA, bufA, semA)
    gather(N_CHUNKS - 1, srcA, bufA, semA)
    wait_scatter(N_CHUNKS - 2, srcB, dstB, bufB, semB)
    wait_scatter(N_CHUNKS - 1, srcA, dstA, bufA, semA)
    plsc.subcore_barrier()

    for t in range(ROWS_PER_TILE // ZROWS):
        r = s * ROWS_PER_TILE + t * ZROWS
        pltpu.sync_copy(acc_sh.at[pl.ds(r, ZROWS)],
                        out_hbm.at[pl.ds(c * ACC_ROWS + r, ZROWS)])


_sc_aggregate = functools.partial(
    pl.kernel,
    out_type=jax.ShapeDtypeStruct((NC * ACC_ROWS, D), jnp.float32),
    mesh=plsc.VectorSubcoreMesh(core_axis_name="c", subcore_axis_name="s"),
    scratch_types=[
        pltpu.VMEM((CHUNK,), jnp.int32),
        pltpu.VMEM((CHUNK,), jnp.int32),
        pltpu.VMEM((CHUNK,), jnp.int32),
        pltpu.VMEM((CHUNK,), jnp.int32),
        pltpu.VMEM((CHUNK, D), jnp.float32),
        pltpu.VMEM((CHUNK, D), jnp.float32),
        pltpu.VMEM((ZROWS, D), jnp.float32),
        pltpu.VMEM_SHARED((ACC_ROWS, D), jnp.float32),
        pltpu.SemaphoreType.DMA,
        pltpu.SemaphoreType.DMA,
    ],
)(_sc_body)


@jax.jit
def kernel(input, edge_index, W, b):
    ei = edge_index.astype(jnp.int32)
    dst = ei[0]
    src = ei[1]

    support = pl.pallas_call(
        _mm_body,
        grid=(10,),
        in_specs=[
            pl.BlockSpec((1000, D), lambda i: (i, 0)),
            pl.BlockSpec((D, D), lambda i: (0, 0)),
        ],
        out_specs=pl.BlockSpec((1000, D), lambda i: (i, 0)),
        out_shape=jax.ShapeDtypeStruct((N_NODES, D), jnp.float32),
    )(input, W)

    partial = _sc_aggregate(support, dst, src)

    out = pl.pallas_call(
        _combine_body,
        grid=(125,),
        in_specs=[
            pl.BlockSpec((80, D), lambda i: (i, 0)),
            pl.BlockSpec((80, D), lambda i: (i + ACC_ROWS // 80, 0)),
            pl.BlockSpec((1, D), lambda i: (0, 0)),
        ],
        out_specs=pl.BlockSpec((80, D), lambda i: (i, 0)),
        out_shape=jax.ShapeDtypeStruct((N_NODES, D), jnp.float32),
    )(partial, partial, b.reshape(1, D))
    return out


# speedup vs baseline: 12.2862x; 1.8089x over previous
import functools

import jax
import jax.numpy as jnp
from jax import lax
from jax.experimental import pallas as pl
from jax.experimental.pallas import tpu as pltpu
from jax.experimental.pallas import tpu_sc as plsc

N_NODES = 10000
N_EDGES = 320000
D = 128

NC = 2
NS = 16
NW = NC * NS
E_PER_TILE = N_EDGES // NW
CHUNK = 128
N_FULL = E_PER_TILE // CHUNK
TAIL = E_PER_TILE - N_FULL * CHUNK
ACC_ROWS = 10240
ROWS_PER_TILE = ACC_ROWS // NS
ZROWS = 128


def _mm_body(x_ref, w_ref, o_ref):
    o_ref[...] = jnp.dot(x_ref[...], w_ref[...],
                         preferred_element_type=jnp.float32)


def _combine_body(p_ref, q_ref, b_ref, o_ref):
    o_ref[...] = p_ref[0] + q_ref[0] + b_ref[...]


def _sc_body(support_hbm, dst_hbm, src_hbm, out_hbm,
             srcA0, srcA1, dstA0, dstA1, srcB0, srcB1, dstB0, dstB1,
             srcT, dstT, bufA, bufB, bufT,
             acc_sh, semA, semB, semIA, semIB):
    c = lax.axis_index("c")
    s = lax.axis_index("s")
    wid = c * NS + s

    zero16 = jnp.zeros((16,), jnp.float32)

    def zstore(i, carry):
        bufA[i // 8, pl.ds((i % 8) * 16, 16)] = zero16
        return carry

    lax.fori_loop(0, ZROWS * (D // 16), zstore, 0)
    for t in range(ROWS_PER_TILE // ZROWS):
        pltpu.sync_copy(bufA,
                        acc_sh.at[pl.ds(s * ROWS_PER_TILE + t * ZROWS, ZROWS)])
    plsc.subcore_barrier()

    ebase = wid * E_PER_TILE

    def make_slot(offs, srcP, dstP, buf, semG, semI):
        def islice(k):
            return pl.ds(ebase + (2 * k + offs) * CHUNK, CHUNK)

        def pre_idx(k, p):
            pltpu.async_copy(src_hbm.at[islice(k)], srcP[p], semI)
            pltpu.async_copy(dst_hbm.at[islice(k)], dstP[p], semI)

        def wait_idx(k, p):
            pltpu.make_async_copy(src_hbm.at[islice(k)], srcP[p], semI).wait()
            pltpu.make_async_copy(dst_hbm.at[islice(k)], dstP[p], semI).wait()

        def gather(p):
            pltpu.async_copy(support_hbm.at[srcP[p]], buf, semG)

        def wait_g(p):
            pltpu.make_async_copy(support_hbm.at[srcP[p]], buf, semG).wait()

        def scatter(p):
            pltpu.sync_copy(buf, acc_sh.at[dstP[p]], add=True)

        def prologue():
            pre_idx(0, 0)
            pre_idx(1, 1)
            wait_idx(0, 0)
            gather(0)

        def step(k, p):
            wait_g(p)
            scatter(p)
            wait_idx(k + 1, 1 - p)
            gather(1 - p)
            pre_idx(k + 2, p)

        def fin():
            wait_g(1)
            scatter(1)
            wait_idx(38, 0)
            gather(0)
            wait_g(0)
            scatter(0)

        return prologue, step, fin

    proA, stepA, finA = make_slot(0, (srcA0, srcA1), (dstA0, dstA1),
                                  bufA, semA, semIA)
    proB, stepB, finB = make_slot(1, (srcB0, srcB1), (dstB0, dstB1),
                                  bufB, semB, semIB)

    proA()
    proB()

    def pipe_body(r, carry):
        k = 2 * r
        stepA(k, 0)
        stepB(k, 0)
        stepA(k + 1, 1)
        stepB(k + 1, 1)
        return carry

    lax.fori_loop(0, 18, pipe_body, 0)
    stepA(36, 0)
    stepB(36, 0)
    finA()
    finB()

    pltpu.sync_copy(src_hbm.at[pl.ds(ebase + N_FULL * CHUNK, TAIL)], srcT)
    pltpu.sync_copy(dst_hbm.at[pl.ds(ebase + N_FULL * CHUNK, TAIL)], dstT)
    pltpu.async_copy(support_hbm.at[srcT], bufT, semA).wait()
    pltpu.sync_copy(bufT, acc_sh.at[dstT], add=True)
    plsc.subcore_barrier()

    for t in range(ROWS_PER_TILE // ZROWS):
        r = s * ROWS_PER_TILE + t * ZROWS
        pltpu.sync_copy(acc_sh.at[pl.ds(r, ZROWS)],
                        out_hbm.at[pl.ds(c * ACC_ROWS + r, ZROWS)])


_sc_aggregate = functools.partial(
    pl.kernel,
    out_type=jax.ShapeDtypeStruct((NC * ACC_ROWS, D), jnp.float32),
    mesh=plsc.VectorSubcoreMesh(core_axis_name="c", subcore_axis_name="s"),
    scratch_types=[
        pltpu.VMEM((CHUNK,), jnp.int32),
        pltpu.VMEM((CHUNK,), jnp.int32),
        pltpu.VMEM((CHUNK,), jnp.int32),
        pltpu.VMEM((CHUNK,), jnp.int32),
        pltpu.VMEM((CHUNK,), jnp.int32),
        pltpu.VMEM((CHUNK,), jnp.int32),
        pltpu.VMEM((CHUNK,), jnp.int32),
        pltpu.VMEM((CHUNK,), jnp.int32),
        pltpu.VMEM((TAIL,), jnp.int32),
        pltpu.VMEM((TAIL,), jnp.int32),
        pltpu.VMEM((CHUNK, D), jnp.float32),
        pltpu.VMEM((CHUNK, D), jnp.float32),
        pltpu.VMEM((TAIL, D), jnp.float32),
        pltpu.VMEM_SHARED((ACC_ROWS, D), jnp.float32),
        pltpu.SemaphoreType.DMA,
        pltpu.SemaphoreType.DMA,
        pltpu.SemaphoreType.DMA,
        pltpu.SemaphoreType.DMA,
    ],
)(_sc_body)


@jax.jit
def kernel(input, edge_index, W, b):
    ei = edge_index.astype(jnp.int32)
    dst = ei[0]
    src = ei[1]

    support = pl.pallas_call(
        _mm_body,
        grid=(10,),
        in_specs=[
            pl.BlockSpec((1000, D), lambda i: (i, 0)),
            pl.BlockSpec((D, D), lambda i: (0, 0)),
        ],
        out_specs=pl.BlockSpec((1000, D), lambda i: (i, 0)),
        out_shape=jax.ShapeDtypeStruct((N_NODES, D), jnp.float32),
    )(input, W)

    partial = _sc_aggregate(support, dst, src)

    partial3 = partial.reshape(NC, ACC_ROWS, D)
    out = pl.pallas_call(
        _combine_body,
        grid=(10,),
        in_specs=[
            pl.BlockSpec((1, 1000, D), lambda i: (0, i, 0)),
            pl.BlockSpec((1, 1000, D), lambda i: (1, i, 0)),
            pl.BlockSpec((1, D), lambda i: (0, 0)),
        ],
        out_specs=pl.BlockSpec((1000, D), lambda i: (i, 0)),
        out_shape=jax.ShapeDtypeStruct((N_NODES, D), jnp.float32),
    )(partial3, partial3, b.reshape(1, D))
    return out


# speedup vs baseline: 12.9650x; 1.0552x over previous
import functools

import jax
import jax.numpy as jnp
from jax import lax
from jax.experimental import pallas as pl
from jax.experimental.pallas import tpu as pltpu
from jax.experimental.pallas import tpu_sc as plsc

N_NODES = 10000
N_EDGES = 320000
D = 128

NC = 2
NS = 16
NW = NC * NS
E_PER_TILE = N_EDGES // NW
CHUNK = 128
N_FULL = E_PER_TILE // CHUNK
TAIL = E_PER_TILE - N_FULL * CHUNK
ACC_ROWS = 10240
ROWS_PER_TILE = ACC_ROWS // NS
ZROWS = 128


def _mm_combine_body(p_ref, q_ref, w_ref, b_ref, o_ref):
    o_ref[...] = jnp.dot(p_ref[0] + q_ref[0], w_ref[...],
                         preferred_element_type=jnp.float32) + b_ref[...]


def _sc_body(support_hbm, dst_hbm, src_hbm, out_hbm,
             srcA0, srcA1, dstA0, dstA1, srcB0, srcB1, dstB0, dstB1,
             srcT, dstT, bufA, bufB, bufT,
             acc_sh, semA, semB, semIA, semIB):
    c = lax.axis_index("c")
    s = lax.axis_index("s")
    wid = c * NS + s

    zero16 = jnp.zeros((16,), jnp.float32)

    def zstore(i, carry):
        bufA[i // 8, pl.ds((i % 8) * 16, 16)] = zero16
        return carry

    lax.fori_loop(0, ZROWS * (D // 16), zstore, 0)
    for t in range(ROWS_PER_TILE // ZROWS):
        pltpu.sync_copy(bufA,
                        acc_sh.at[pl.ds(s * ROWS_PER_TILE + t * ZROWS, ZROWS)])
    plsc.subcore_barrier()

    ebase = wid * E_PER_TILE

    def make_slot(offs, srcP, dstP, buf, semG, semI):
        def islice(k):
            return pl.ds(ebase + (2 * k + offs) * CHUNK, CHUNK)

        def pre_idx(k, p):
            pltpu.async_copy(src_hbm.at[islice(k)], srcP[p], semI)
            pltpu.async_copy(dst_hbm.at[islice(k)], dstP[p], semI)

        def wait_idx(k, p):
            pltpu.make_async_copy(src_hbm.at[islice(k)], srcP[p], semI).wait()
            pltpu.make_async_copy(dst_hbm.at[islice(k)], dstP[p], semI).wait()

        def gather(p):
            pltpu.async_copy(support_hbm.at[srcP[p]], buf, semG)

        def wait_g(p):
            pltpu.make_async_copy(support_hbm.at[srcP[p]], buf, semG).wait()

        def scatter(p):
            pltpu.sync_copy(buf, acc_sh.at[dstP[p]], add=True)

        def prologue():
            pre_idx(0, 0)
            pre_idx(1, 1)
            wait_idx(0, 0)
            gather(0)

        def step(k, p):
            wait_g(p)
            scatter(p)
            wait_idx(k + 1, 1 - p)
            gather(1 - p)
            pre_idx(k + 2, p)

        def fin():
            wait_g(1)
            scatter(1)
            wait_idx(38, 0)
            gather(0)
            wait_g(0)
            scatter(0)

        return prologue, step, fin

    proA, stepA, finA = make_slot(0, (srcA0, srcA1), (dstA0, dstA1),
                                  bufA, semA, semIA)
    proB, stepB, finB = make_slot(1, (srcB0, srcB1), (dstB0, dstB1),
                                  bufB, semB, semIB)

    proA()
    proB()

    def pipe_body(r, carry):
        k = 2 * r
        stepA(k, 0)
        stepB(k, 0)
        stepA(k + 1, 1)
        stepB(k + 1, 1)
        return carry

    lax.fori_loop(0, 18, pipe_body, 0)
    stepA(36, 0)
    stepB(36, 0)
    finA()
    finB()

    pltpu.sync_copy(src_hbm.at[pl.ds(ebase + N_FULL * CHUNK, TAIL)], srcT)
    pltpu.sync_copy(dst_hbm.at[pl.ds(ebase + N_FULL * CHUNK, TAIL)], dstT)
    pltpu.async_copy(support_hbm.at[srcT], bufT, semA).wait()
    pltpu.sync_copy(bufT, acc_sh.at[dstT], add=True)
    plsc.subcore_barrier()

    for t in range(ROWS_PER_TILE // ZROWS):
        r = s * ROWS_PER_TILE + t * ZROWS
        pltpu.sync_copy(acc_sh.at[pl.ds(r, ZROWS)],
                        out_hbm.at[pl.ds(c * ACC_ROWS + r, ZROWS)])


_sc_aggregate = functools.partial(
    pl.kernel,
    out_type=jax.ShapeDtypeStruct((NC * ACC_ROWS, D), jnp.float32),
    mesh=plsc.VectorSubcoreMesh(core_axis_name="c", subcore_axis_name="s"),
    scratch_types=[
        pltpu.VMEM((CHUNK,), jnp.int32),
        pltpu.VMEM((CHUNK,), jnp.int32),
        pltpu.VMEM((CHUNK,), jnp.int32),
        pltpu.VMEM((CHUNK,), jnp.int32),
        pltpu.VMEM((CHUNK,), jnp.int32),
        pltpu.VMEM((CHUNK,), jnp.int32),
        pltpu.VMEM((CHUNK,), jnp.int32),
        pltpu.VMEM((CHUNK,), jnp.int32),
        pltpu.VMEM((TAIL,), jnp.int32),
        pltpu.VMEM((TAIL,), jnp.int32),
        pltpu.VMEM((CHUNK, D), jnp.float32),
        pltpu.VMEM((CHUNK, D), jnp.float32),
        pltpu.VMEM((TAIL, D), jnp.float32),
        pltpu.VMEM_SHARED((ACC_ROWS, D), jnp.float32),
        pltpu.SemaphoreType.DMA,
        pltpu.SemaphoreType.DMA,
        pltpu.SemaphoreType.DMA,
        pltpu.SemaphoreType.DMA,
    ],
)(_sc_body)


@jax.jit
def kernel(input, edge_index, W, b):
    ei = edge_index.astype(jnp.int32)
    dst = ei[0]
    src = ei[1]

    partial = _sc_aggregate(input, dst, src)

    partial3 = partial.reshape(NC, ACC_ROWS, D)
    out = pl.pallas_call(
        _mm_combine_body,
        grid=(10,),
        in_specs=[
            pl.BlockSpec((1, 1000, D), lambda i: (0, i, 0)),
            pl.BlockSpec((1, 1000, D), lambda i: (1, i, 0)),
            pl.BlockSpec((D, D), lambda i: (0, 0)),
            pl.BlockSpec((1, D), lambda i: (0, 0)),
        ],
        out_specs=pl.BlockSpec((1000, D), lambda i: (i, 0)),
        out_shape=jax.ShapeDtypeStruct((N_NODES, D), jnp.float32),
    )(partial3, partial3, W, b.reshape(1, D))
    return out


# speedup vs baseline: 13.1343x; 1.0131x over previous
import functools

import jax
import jax.numpy as jnp
from jax import lax
from jax.experimental import pallas as pl
from jax.experimental.pallas import tpu as pltpu
from jax.experimental.pallas import tpu_sc as plsc

N_NODES = 10000
N_EDGES = 320000
D = 128

NC = 2
NS = 16
NW = NC * NS
E_PER_TILE = N_EDGES // NW
CHUNK = 128
N_FULL = E_PER_TILE // CHUNK
TAIL = E_PER_TILE - N_FULL * CHUNK
ACC_ROWS = 10240
ROWS_PER_TILE = ACC_ROWS // NS
ZROWS = 64
WROWS = 128


def _mm_combine_body(p_ref, q_ref, w_ref, b_ref, o_ref):
    o_ref[...] = jnp.dot(p_ref[0] + q_ref[0], w_ref[...],
                         preferred_element_type=jnp.float32) + b_ref[...]


def _sc_body(support_hbm, dst_hbm, src_hbm, out_hbm,
             srcA0, srcA1, dstA0, dstA1, srcB0, srcB1, dstB0, dstB1,
             srcT, dstT, bufA, bufB, bufT, zbuf_v,
             acc_sh, semA, semB, semIA, semIB):
    c = lax.axis_index("c")
    s = lax.axis_index("s")
    wid = c * NS + s

    ebase = wid * E_PER_TILE

    def make_slot(offs, srcP, dstP, buf, semG, semI):
        def islice(k):
            return pl.ds(ebase + (2 * k + offs) * CHUNK, CHUNK)

        def pre_idx(k, p):
            pltpu.async_copy(src_hbm.at[islice(k)], srcP[p], semI)
            pltpu.async_copy(dst_hbm.at[islice(k)], dstP[p], semI)

        def wait_idx(k, p):
            pltpu.make_async_copy(src_hbm.at[islice(k)], srcP[p], semI).wait()
            pltpu.make_async_copy(dst_hbm.at[islice(k)], dstP[p], semI).wait()

        def gather(p):
            pltpu.async_copy(support_hbm.at[srcP[p]], buf, semG)

        def wait_g(p):
            pltpu.make_async_copy(support_hbm.at[srcP[p]], buf, semG).wait()

        def scatter(p):
            pltpu.sync_copy(buf, acc_sh.at[dstP[p]], add=True)

        def prologue():
            pre_idx(0, 0)
            pre_idx(1, 1)
            wait_idx(0, 0)
            gather(0)

        def step(k, p):
            wait_g(p)
            scatter(p)
            wait_idx(k + 1, 1 - p)
            gather(1 - p)
            pre_idx(k + 2, p)

        def fin():
            wait_g(1)
            scatter(1)
            wait_idx(38, 0)
            gather(0)
            wait_g(0)
            scatter(0)

        return prologue, step, fin

    proA, stepA, finA = make_slot(0, (srcA0, srcA1), (dstA0, dstA1),
                                  bufA, semA, semIA)
    proB, stepB, finB = make_slot(1, (srcB0, srcB1), (dstB0, dstB1),
                                  bufB, semB, semIB)

    proA()
    proB()

    zero16 = jnp.zeros((16,), jnp.float32)

    def zstore(i, carry):
        zbuf_v[i // 8, pl.ds((i % 8) * 16, 16)] = zero16
        return carry

    lax.fori_loop(0, ZROWS * (D // 16), zstore, 0)
    for t in range(ROWS_PER_TILE // ZROWS):
        pltpu.sync_copy(zbuf_v,
                        acc_sh.at[pl.ds(s * ROWS_PER_TILE + t * ZROWS, ZROWS)])
    plsc.subcore_barrier()

    def pipe_body(r, carry):
        k = 2 * r
        stepA(k, 0)
        stepB(k, 0)
        stepA(k + 1, 1)
        stepB(k + 1, 1)
        return carry

    lax.fori_loop(0, 18, pipe_body, 0)
    stepA(36, 0)
    stepB(36, 0)
    finA()
    finB()

    pltpu.sync_copy(src_hbm.at[pl.ds(ebase + N_FULL * CHUNK, TAIL)], srcT)
    pltpu.sync_copy(dst_hbm.at[pl.ds(ebase + N_FULL * CHUNK, TAIL)], dstT)
    pltpu.async_copy(support_hbm.at[srcT], bufT, semA).wait()
    pltpu.sync_copy(bufT, acc_sh.at[dstT], add=True)
    plsc.subcore_barrier()

    cps = []
    for t in range(ROWS_PER_TILE // WROWS):
        r = s * ROWS_PER_TILE + t * WROWS
        cps.append(pltpu.async_copy(acc_sh.at[pl.ds(r, WROWS)],
                                    out_hbm.at[pl.ds(c * ACC_ROWS + r, WROWS)],
                                    semA))
    for cp in cps:
        cp.wait()


_sc_aggregate = functools.partial(
    pl.kernel,
    out_type=jax.ShapeDtypeStruct((NC * ACC_ROWS, D), jnp.float32),
    mesh=plsc.VectorSubcoreMesh(core_axis_name="c", subcore_axis_name="s"),
    scratch_types=[
        pltpu.VMEM((CHUNK,), jnp.int32),
        pltpu.VMEM((CHUNK,), jnp.int32),
        pltpu.VMEM((CHUNK,), jnp.int32),
        pltpu.VMEM((CHUNK,), jnp.int32),
        pltpu.VMEM((CHUNK,), jnp.int32),
        pltpu.VMEM((CHUNK,), jnp.int32),
        pltpu.VMEM((CHUNK,), jnp.int32),
        pltpu.VMEM((CHUNK,), jnp.int32),
        pltpu.VMEM((TAIL,), jnp.int32),
        pltpu.VMEM((TAIL,), jnp.int32),
        pltpu.VMEM((CHUNK, D), jnp.float32),
        pltpu.VMEM((CHUNK, D), jnp.float32),
        pltpu.VMEM((TAIL, D), jnp.float32),
        pltpu.VMEM((ZROWS, D), jnp.float32),
        pltpu.VMEM_SHARED((ACC_ROWS, D), jnp.float32),
        pltpu.SemaphoreType.DMA,
        pltpu.SemaphoreType.DMA,
        pltpu.SemaphoreType.DMA,
        pltpu.SemaphoreType.DMA,
    ],
)(_sc_body)


@jax.jit
def kernel(input, edge_index, W, b):
    ei = edge_index.astype(jnp.int32)
    dst = ei[0]
    src = ei[1]

    partial = _sc_aggregate(input, dst, src)

    partial3 = partial.reshape(NC, ACC_ROWS, D)
    out = pl.pallas_call(
        _mm_combine_body,
        grid=(10,),
        in_specs=[
            pl.BlockSpec((1, 1000, D), lambda i: (0, i, 0)),
            pl.BlockSpec((1, 1000, D), lambda i: (1, i, 0)),
            pl.BlockSpec((D, D), lambda i: (0, 0)),
            pl.BlockSpec((1, D), lambda i: (0, 0)),
        ],
        out_specs=pl.BlockSpec((1000, D), lambda i: (i, 0)),
        out_shape=jax.ShapeDtypeStruct((N_NODES, D), jnp.float32),
    )(partial3, partial3, W, b.reshape(1, D))
    return out
